# baseline (speedup 1.0000x reference)
import jax
import jax.numpy as jnp
from jax.experimental import pallas as pl

N = 16384
C = 1000
BR = 512
GRID = N // BR


def _onehot_block(idx_ref, out_ref):
    idx = idx_ref[0, 0, :]
    cols = jax.lax.broadcasted_iota(jnp.int32, (BR, C), 1)
    out_ref[...] = jnp.where(idx[:, None] == cols, jnp.float32(1.0),
                             jnp.float32(0.0))


def kernel(inputs):
    idx3 = inputs.astype(jnp.int32).reshape(GRID, 1, BR)
    out = pl.pallas_call(
        _onehot_block,
        grid=(GRID,),
        in_specs=[pl.BlockSpec((1, 1, BR), lambda i: (i, 0, 0))],
        out_specs=pl.BlockSpec((BR, C), lambda i: (i, 0)),
        out_shape=jax.ShapeDtypeStruct((N, C), jnp.float32),
    )(idx3)
    return out


# speedup vs baseline: 1.0740x; 1.0740x over previous
import jax
import jax.numpy as jnp
from jax.experimental import pallas as pl

N = 16384
C = 1000
BR = 2048
GRID = N // BR


def _onehot_block(idx_ref, out_ref):
    idx = idx_ref[0, 0, :]
    cols = jax.lax.broadcasted_iota(jnp.int32, (BR, C), 1)
    out_ref[...] = jnp.where(idx[:, None] == cols, jnp.float32(1.0),
                             jnp.float32(0.0))


def kernel(inputs):
    idx3 = inputs.astype(jnp.int32).reshape(GRID, 1, BR)
    out = pl.pallas_call(
        _onehot_block,
        grid=(GRID,),
        in_specs=[pl.BlockSpec((1, 1, BR), lambda i: (i, 0, 0))],
        out_specs=pl.BlockSpec((BR, C), lambda i: (i, 0)),
        out_shape=jax.ShapeDtypeStruct((N, C), jnp.float32),
    )(idx3)
    return out


# speedup vs baseline: 1.0785x; 1.0042x over previous
import jax
import jax.numpy as jnp
from jax import lax
from jax.experimental import pallas as pl
from jax.experimental.pallas import tpu as pltpu

N = 16384
C = 1000
BR = 512
GRID = N // BR
NBUF = 4


def _onehot_block(idx_ref, out_ref, buf, sem):
    i = pl.program_id(0)
    slot = lax.rem(i, NBUF)

    @pl.when(i >= NBUF)
    def _():
        pltpu.make_async_copy(
            buf.at[slot], out_ref.at[pl.ds(0, BR), :], sem.at[slot]
        ).wait()

    idx = idx_ref[0, 0, :]
    cols = jax.lax.broadcasted_iota(jnp.int32, (BR, C), 1)
    buf[slot] = jnp.where(idx[:, None] == cols, jnp.float32(1.0),
                          jnp.float32(0.0))
    pltpu.make_async_copy(
        buf.at[slot], out_ref.at[pl.ds(i * BR, BR), :], sem.at[slot]
    ).start()

    @pl.when(i == GRID - 1)
    def _():
        for s in range(NBUF):
            pltpu.make_async_copy(
                buf.at[s], out_ref.at[pl.ds(0, BR), :], sem.at[s]
            ).wait()


def kernel(inputs):
    idx3 = inputs.astype(jnp.int32).reshape(GRID, 1, BR)
    out = pl.pallas_call(
        _onehot_block,
        grid=(GRID,),
        in_specs=[pl.BlockSpec((1, 1, BR), lambda i: (i, 0, 0))],
        out_specs=pl.BlockSpec(memory_space=pltpu.MemorySpace.HBM),
        out_shape=jax.ShapeDtypeStruct((N, C), jnp.float32),
        scratch_shapes=[
            pltpu.VMEM((NBUF, BR, C), jnp.float32),
            pltpu.SemaphoreType.DMA((NBUF,)),
        ],
    )(idx3)
    return out
